# baseline (speedup 1.0000x reference)
import functools

import jax
import jax.numpy as jnp
from jax import lax
from jax.experimental import pallas as pl
from jax.experimental.pallas import tpu as pltpu
from jax.experimental.pallas import tpu_sc as plsc

_ROWS = 8192
_D = 1024


def _sc_copy_fn():
    info = plsc.get_sparse_core_info()
    nc, ns = info.num_cores, info.num_subcores
    nw = nc * ns
    rows_per_w = _ROWS // nw

    mesh = plsc.VectorSubcoreMesh(core_axis_name="c", subcore_axis_name="s")

    @functools.partial(
        pl.kernel,
        mesh=mesh,
        out_type=jax.ShapeDtypeStruct((_ROWS, _D), jnp.float32),
        scratch_types=[pltpu.SemaphoreType.DMA],
    )
    def sc_copy(wpe_hbm, out_hbm, sem):
        wid = lax.axis_index("s") * nc + lax.axis_index("c")
        base = wid * rows_per_w
        pltpu.async_copy(
            wpe_hbm.at[pl.ds(base, rows_per_w)],
            out_hbm.at[pl.ds(base, rows_per_w)],
            sem,
        ).wait()

    return sc_copy


def kernel(x, wpe):
    del x
    return _sc_copy_fn()(wpe)


# speedup vs baseline: 23.2068x; 23.2068x over previous
import functools

import jax
import jax.numpy as jnp
from jax import lax
from jax.experimental import pallas as pl
from jax.experimental.pallas import tpu as pltpu
from jax.experimental.pallas import tpu_sc as plsc

_ROWS = 8192
_D = 1024
_CH = 32


def _sc_copy_fn():
    info = plsc.get_sparse_core_info()
    nc, ns = info.num_cores, info.num_subcores
    rows_per_w = _ROWS // (nc * ns)
    n_chunks = rows_per_w // _CH

    mesh = plsc.VectorSubcoreMesh(core_axis_name="c", subcore_axis_name="s")

    @functools.partial(
        pl.kernel,
        mesh=mesh,
        out_type=jax.ShapeDtypeStruct((_ROWS, _D), jnp.float32),
        scratch_types=[
            pltpu.VMEM((_CH, _D), jnp.float32),
            pltpu.VMEM((_CH, _D), jnp.float32),
            pltpu.SemaphoreType.DMA,
            pltpu.SemaphoreType.DMA,
        ],
    )
    def sc_copy(wpe_hbm, out_hbm, buf0, buf1, in_sem, out_sem):
        wid = lax.axis_index("s") * nc + lax.axis_index("c")
        base = wid * rows_per_w
        bufs = (buf0, buf1)
        gathers = [
            pltpu.async_copy(wpe_hbm.at[pl.ds(base, _CH)], buf0, in_sem)
        ]
        scatters = []
        for i in range(n_chunks):
            gathers[i].wait()
            scatters.append(
                pltpu.async_copy(
                    bufs[i % 2], out_hbm.at[pl.ds(base + i * _CH, _CH)], out_sem
                )
            )
            if i + 1 < n_chunks:
                if i >= 1:
                    scatters[i - 1].wait()
                gathers.append(
                    pltpu.async_copy(
                        wpe_hbm.at[pl.ds(base + (i + 1) * _CH, _CH)],
                        bufs[(i + 1) % 2],
                        in_sem,
                    )
                )
        scatters[-2].wait()
        scatters[-1].wait()

    return sc_copy


def kernel(x, wpe):
    del x
    return _sc_copy_fn()(wpe)


# speedup vs baseline: 24.6264x; 1.0612x over previous
import functools

import jax
import jax.numpy as jnp
from jax import lax
from jax.experimental import pallas as pl
from jax.experimental.pallas import tpu as pltpu
from jax.experimental.pallas import tpu_sc as plsc

_ROWS = 8192
_D = 1024
_CH = 32
_NBUF = 3


def _sc_copy_fn():
    info = plsc.get_sparse_core_info()
    nc, ns = info.num_cores, info.num_subcores
    rows_per_w = _ROWS // (nc * ns)
    n_chunks = rows_per_w // _CH

    mesh = plsc.VectorSubcoreMesh(core_axis_name="c", subcore_axis_name="s")

    @functools.partial(
        pl.kernel,
        mesh=mesh,
        out_type=jax.ShapeDtypeStruct((_ROWS, _D), jnp.float32),
        scratch_types=[pltpu.VMEM((_CH, _D), jnp.float32)] * _NBUF
        + [pltpu.SemaphoreType.DMA, pltpu.SemaphoreType.DMA],
    )
    def sc_copy(wpe_hbm, out_hbm, *bufs_and_sems):
        bufs = bufs_and_sems[:_NBUF]
        in_sem, out_sem = bufs_and_sems[_NBUF:]
        wid = lax.axis_index("s") * nc + lax.axis_index("c")
        base = wid * rows_per_w
        gathers, scatters = [], []
        for i in range(min(_NBUF, n_chunks)):
            gathers.append(
                pltpu.async_copy(
                    wpe_hbm.at[pl.ds(base + i * _CH, _CH)], bufs[i], in_sem
                )
            )
        for i in range(n_chunks):
            gathers[i].wait()
            scatters.append(
                pltpu.async_copy(
                    bufs[i % _NBUF], out_hbm.at[pl.ds(base + i * _CH, _CH)], out_sem
                )
            )
            j = i + _NBUF
            if j < n_chunks:
                scatters[j - _NBUF].wait()
                gathers.append(
                    pltpu.async_copy(
                        wpe_hbm.at[pl.ds(base + j * _CH, _CH)],
                        bufs[j % _NBUF],
                        in_sem,
                    )
                )
        for i in range(max(0, n_chunks - _NBUF), n_chunks):
            scatters[i].wait()

    return sc_copy


def kernel(x, wpe):
    del x
    return _sc_copy_fn()(wpe)
